# baseline (speedup 1.0000x reference)
import functools

import jax
import jax.numpy as jnp
from jax import lax
from jax.experimental import pallas as pl
from jax.experimental.pallas import tpu as pltpu
from jax.experimental.pallas import tpu_sc as plsc

B, S, D = 16, 4096, 1024
BS = 128
NSB = S // BS
NEG_INF = float("-inf")


def _sc_speaker_body(sot_hbm, seq_hbm, spk_hbm, sot_v, row_v):
    wid = lax.axis_index("s") * 2 + lax.axis_index("c")
    b = wid % B
    pltpu.sync_copy(sot_hbm.at[pl.ds(b * S, S)], sot_v)
    iota16 = lax.iota(jnp.int32, 16)

    accs = [jnp.full((16,), S, jnp.int32) for _ in range(8)]
    for i in range(S // 16):
        chunk = sot_v[pl.ds(i * 16, 16)]
        accs[i % 8] = jnp.minimum(
            accs[i % 8], jnp.where(chunk != 0, jnp.int32(i * 16) + iota16, S))
    while len(accs) > 1:
        accs = [jnp.minimum(a, c) for a, c in zip(accs[::2], accs[1::2])]
    best = accs[0]
    vals = [best[j] for j in range(16)]
    while len(vals) > 1:
        vals = [jnp.minimum(a, c) for a, c in zip(vals[::2], vals[1::2])]
    fp = vals[0]
    fp = jnp.where(fp >= S, 0, fp)
    pltpu.sync_copy(seq_hbm.at[pl.ds(b * S + fp, 1)], row_v)
    pltpu.sync_copy(row_v, spk_hbm.at[pl.ds(wid, 1)])


def _cumsum_lastdim(x):
    k = 1
    while k < S:
        shifted = jnp.concatenate(
            [jnp.zeros((B, k), x.dtype), x[:, : S - k]], axis=1)
        x = x + shifted
        k *= 2
    return x


def _stream_body(seq_ref, spk_ref, sot_ref, labels_ref,
                 loss_ref, pred_ref, dot_acc, nsq_acc):
    s = pl.program_id(0)
    x = seq_ref[...]
    spk = spk_ref[...][:, None, :]
    xb = x.astype(jnp.bfloat16).astype(jnp.float32)
    sb = spk.astype(jnp.bfloat16).astype(jnp.float32)
    dot_acc[:, pl.ds(s * BS, BS)] = jnp.sum(xb * sb, axis=2)
    nsq_acc[:, pl.ds(s * BS, BS)] = jnp.sum(x * x, axis=2)

    @pl.when(s == NSB - 1)
    def _finalize():
        dot = dot_acc[...]
        nsq = nsq_acc[...]
        is_sot = sot_ref[...] != 0
        labels = labels_ref[...]

        cs = _cumsum_lastdim(is_sot.astype(jnp.int32))
        spk_nsq = jnp.sum(spk * spk, axis=2)
        denom = jnp.maximum(jnp.sqrt(nsq) * jnp.sqrt(spk_nsq), 1e-6)
        sim = dot / denom
        remain = is_sot & (cs >= 2)
        simm = jnp.where(remain, sim, NEG_INF)

        m = jnp.max(simm, axis=1, keepdims=True)
        lse = m + jnp.log(jnp.sum(jnp.exp(simm - m), axis=1, keepdims=True))

        lmask = is_sot & (cs == labels + 2)
        has_l = jnp.any(lmask, axis=1, keepdims=True)
        val_l = jnp.sum(jnp.where(lmask, simm, 0.0), axis=1, keepdims=True)
        logp = jnp.where(has_l, val_l, simm[:, 0:1]) - lse
        loss_ref[...] = jnp.mean(-logp)[None, None]

        iota = lax.broadcasted_iota(jnp.int32, (B, S), 1)
        ppos = jnp.min(jnp.where(simm == m, iota, S), axis=1, keepdims=True)
        ppos = jnp.where(ppos == S, 0, ppos)
        pcs = jnp.sum(jnp.where(iota == ppos, cs, 0), axis=1, keepdims=True)
        pred_ref[...] = pcs - 2


def kernel(sequence_output, sot_positions, labels):
    sot_positions = sot_positions.astype(jnp.int32)

    sc_speaker = functools.partial(
        pl.kernel,
        mesh=plsc.VectorSubcoreMesh(core_axis_name="c", subcore_axis_name="s"),
        out_type=jax.ShapeDtypeStruct((2 * B, D), jnp.float32),
        scratch_types=[
            pltpu.VMEM((S,), jnp.int32),
            pltpu.VMEM((1, D), jnp.float32),
        ],
    )(_sc_speaker_body)
    speakers = sc_speaker(
        sot_positions.reshape(B * S),
        sequence_output.reshape(B * S, D))

    loss, pred = pl.pallas_call(
        _stream_body,
        grid=(NSB,),
        in_specs=[
            pl.BlockSpec((B, BS, D), lambda s: (0, s, 0)),
            pl.BlockSpec((B, D), lambda s: (0, 0)),
            pl.BlockSpec((B, S), lambda s: (0, 0)),
            pl.BlockSpec((B, 1), lambda s: (0, 0)),
        ],
        out_specs=[
            pl.BlockSpec((1, 1), lambda s: (0, 0)),
            pl.BlockSpec((B, 1), lambda s: (0, 0)),
        ],
        out_shape=[
            jax.ShapeDtypeStruct((1, 1), jnp.float32),
            jax.ShapeDtypeStruct((B, 1), jnp.int32),
        ],
        scratch_shapes=[
            pltpu.VMEM((B, S), jnp.float32),
            pltpu.VMEM((B, S), jnp.float32),
        ],
    )(sequence_output, speakers, sot_positions,
      labels.astype(jnp.int32).reshape(B, 1))

    return (loss[0, 0], pred.reshape(B), labels.astype(jnp.int32))


# speedup vs baseline: 1.0463x; 1.0463x over previous
import functools

import jax
import jax.numpy as jnp
from jax import lax
from jax.experimental import pallas as pl
from jax.experimental.pallas import tpu as pltpu
from jax.experimental.pallas import tpu_sc as plsc

B, S, D = 16, 4096, 1024
BS = 256
NSB = S // BS
NEG_INF = float("-inf")


def _sc_speaker_body(sot_hbm, seq_hbm, spk_hbm, sot_v, row_v):
    wid = lax.axis_index("s") * 2 + lax.axis_index("c")
    b = wid % B
    pltpu.sync_copy(sot_hbm.at[pl.ds(b * S, S)], sot_v)
    iota16 = lax.iota(jnp.int32, 16)

    accs = [jnp.full((16,), S, jnp.int32) for _ in range(8)]
    for i in range(S // 16):
        chunk = sot_v[pl.ds(i * 16, 16)]
        accs[i % 8] = jnp.minimum(
            accs[i % 8], jnp.where(chunk != 0, jnp.int32(i * 16) + iota16, S))
    while len(accs) > 1:
        accs = [jnp.minimum(a, c) for a, c in zip(accs[::2], accs[1::2])]
    best = accs[0]
    vals = [best[j] for j in range(16)]
    while len(vals) > 1:
        vals = [jnp.minimum(a, c) for a, c in zip(vals[::2], vals[1::2])]
    fp = vals[0]
    fp = jnp.where(fp >= S, 0, fp)
    pltpu.sync_copy(seq_hbm.at[pl.ds(b * S + fp, 1)], row_v)
    pltpu.sync_copy(row_v, spk_hbm.at[pl.ds(wid, 1)])


def _cumsum_lastdim(x):
    k = 1
    while k < S:
        shifted = jnp.concatenate(
            [jnp.zeros((B, k), x.dtype), x[:, : S - k]], axis=1)
        x = x + shifted
        k *= 2
    return x


def _stream_body(seq_ref, spk_ref, sot_ref, labels_ref,
                 loss_ref, pred_ref, dot_acc, nsq_acc):
    s = pl.program_id(0)
    x = seq_ref[...]
    spk = spk_ref[...][:, None, :]
    xb = x.astype(jnp.bfloat16).astype(jnp.float32)
    sb = spk.astype(jnp.bfloat16).astype(jnp.float32)
    dot_acc[:, pl.ds(s * BS, BS)] = jnp.sum(xb * sb, axis=2)
    nsq_acc[:, pl.ds(s * BS, BS)] = jnp.sum(x * x, axis=2)

    @pl.when(s == NSB - 1)
    def _finalize():
        dot = dot_acc[...]
        nsq = nsq_acc[...]
        is_sot = sot_ref[...] != 0
        labels = labels_ref[...]

        cs = _cumsum_lastdim(is_sot.astype(jnp.int32))
        spk_nsq = jnp.sum(spk * spk, axis=2)
        denom = jnp.maximum(jnp.sqrt(nsq) * jnp.sqrt(spk_nsq), 1e-6)
        sim = dot / denom
        remain = is_sot & (cs >= 2)
        simm = jnp.where(remain, sim, NEG_INF)

        m = jnp.max(simm, axis=1, keepdims=True)
        lse = m + jnp.log(jnp.sum(jnp.exp(simm - m), axis=1, keepdims=True))

        lmask = is_sot & (cs == labels + 2)
        has_l = jnp.any(lmask, axis=1, keepdims=True)
        val_l = jnp.sum(jnp.where(lmask, simm, 0.0), axis=1, keepdims=True)
        logp = jnp.where(has_l, val_l, simm[:, 0:1]) - lse
        loss_ref[...] = jnp.mean(-logp)[None, None]

        iota = lax.broadcasted_iota(jnp.int32, (B, S), 1)
        ppos = jnp.min(jnp.where(simm == m, iota, S), axis=1, keepdims=True)
        ppos = jnp.where(ppos == S, 0, ppos)
        pcs = jnp.sum(jnp.where(iota == ppos, cs, 0), axis=1, keepdims=True)
        pred_ref[...] = pcs - 2


def kernel(sequence_output, sot_positions, labels):
    sot_positions = sot_positions.astype(jnp.int32)

    sc_speaker = functools.partial(
        pl.kernel,
        mesh=plsc.VectorSubcoreMesh(core_axis_name="c", subcore_axis_name="s"),
        out_type=jax.ShapeDtypeStruct((2 * B, D), jnp.float32),
        scratch_types=[
            pltpu.VMEM((S,), jnp.int32),
            pltpu.VMEM((1, D), jnp.float32),
        ],
    )(_sc_speaker_body)
    speakers = sc_speaker(
        sot_positions.reshape(B * S),
        sequence_output.reshape(B * S, D))

    loss, pred = pl.pallas_call(
        _stream_body,
        grid=(NSB,),
        in_specs=[
            pl.BlockSpec((B, BS, D), lambda s: (0, s, 0)),
            pl.BlockSpec((B, D), lambda s: (0, 0)),
            pl.BlockSpec((B, S), lambda s: (0, 0)),
            pl.BlockSpec((B, 1), lambda s: (0, 0)),
        ],
        out_specs=[
            pl.BlockSpec((1, 1), lambda s: (0, 0)),
            pl.BlockSpec((B, 1), lambda s: (0, 0)),
        ],
        out_shape=[
            jax.ShapeDtypeStruct((1, 1), jnp.float32),
            jax.ShapeDtypeStruct((B, 1), jnp.int32),
        ],
        scratch_shapes=[
            pltpu.VMEM((B, S), jnp.float32),
            pltpu.VMEM((B, S), jnp.float32),
        ],
    )(sequence_output, speakers, sot_positions,
      labels.astype(jnp.int32).reshape(B, 1))

    return (loss[0, 0], pred.reshape(B), labels.astype(jnp.int32))
